# baseline (speedup 1.0000x reference)
import functools

import jax
import jax.numpy as jnp
from jax import lax
from jax.experimental import pallas as pl
from jax.experimental.pallas import tpu as pltpu
from jax.experimental.pallas import tpu_sc as plsc

N = 10000
E = 320000
D = 128

NC = 2
NS = 16
NW = NC * NS
CB = 128
CH = 80
EP = NW * CH * CB
TOTCH = EP // CB
FC = 1
CPS = TOTCH // NS
SG = 40
NP = 10112
RPS = NP // NS
TRASH = N
RB = 2528

_mesh = plsc.VectorSubcoreMesh(core_axis_name="c", subcore_axis_name="s")


@functools.partial(
    pl.kernel,
    out_type=jax.ShapeDtypeStruct((NC, NP, D), jnp.float32),
    mesh=_mesh,
    scratch_types=[
        pltpu.VMEM((CH, CB), jnp.int32),
        pltpu.VMEM((CB, D), jnp.float32),
        pltpu.VMEM_SHARED((NP, D), jnp.float32),
        pltpu.SemaphoreType.DMA,
    ],
)
def _deg_kernel(dst_hbm, ones_hbm, zero_hbm, out_hbm, dst_v, ones_v, acc, sem):
    cid = lax.axis_index("c")
    sid = lax.axis_index("s")
    wid = sid * NC + cid
    row = pl.ds(sid * RPS, RPS)
    pltpu.sync_copy(zero_hbm.at[row], acc.at[row])
    pltpu.sync_copy(dst_hbm.at[pl.ds(wid * CH, CH)], dst_v)
    pltpu.sync_copy(ones_hbm, ones_v)
    plsc.subcore_barrier()

    GK = 8

    def body(g, carry):
        for i in range(GK):
            pltpu.async_copy(ones_v, acc.at[dst_v.at[g * GK + i]], sem, add=True)
        for i in range(GK):
            pltpu.make_async_copy(ones_v, acc.at[dst_v.at[g * GK + i]], sem).wait()
        return carry

    lax.fori_loop(0, CH // GK, body, 0)
    plsc.subcore_barrier()
    pltpu.sync_copy(acc.at[row], out_hbm.at[cid, row])


@functools.partial(
    pl.kernel,
    out_type=jax.ShapeDtypeStruct((NP, D), jnp.float32),
    mesh=_mesh,
    scratch_types=[
        pltpu.VMEM((SG, CB), jnp.int32),
        pltpu.VMEM((SG, CB), jnp.int32),
        pltpu.VMEM((CB, D), jnp.float32),
        pltpu.VMEM((CB, D), jnp.float32),
        pltpu.VMEM_SHARED((NP, D), jnp.float32),
        pltpu.SemaphoreType.DMA,
        pltpu.SemaphoreType.DMA,
    ],
)
def _agg_kernel(src_hbm, dst_hbm, xs_hbm, zero_hbm, out_hbm,
                src_v, dst_v, buf_a, buf_b, acc, gsem, ssem):
    cid = lax.axis_index("c")
    sid = lax.axis_index("s")
    row = pl.ds(sid * RPS, RPS)
    base = sid * CPS

    def gather(j, buf):
        pltpu.async_copy(xs_hbm.at[src_v.at[j]], buf, gsem)

    def gather_wait(buf):
        pltpu.make_async_copy(xs_hbm.at[src_v.at[0]], buf, gsem).wait()

    def scat(j, buf):
        pltpu.async_copy(buf, acc.at[dst_v.at[j]], ssem, add=True)

    def scat_wait(buf):
        pltpu.make_async_copy(buf, acc.at[dst_v.at[0]], ssem).wait()

    @pl.when(cid == FC)
    def _():
        pltpu.sync_copy(zero_hbm.at[row], acc.at[row])
        plsc.subcore_barrier()

        def stage(h, carry):
            pltpu.sync_copy(src_hbm.at[pl.ds(base + h * SG, SG)], src_v)
            pltpu.sync_copy(dst_hbm.at[pl.ds(base + h * SG, SG)], dst_v)
            gather(0, buf_a)

            def body(k, inner):
                j0 = 2 * k
                gather_wait(buf_a)

                @pl.when(k > 0)
                def _():
                    scat_wait(buf_b)

                gather(j0 + 1, buf_b)
                scat(j0, buf_a)
                gather_wait(buf_b)
                scat_wait(buf_a)

                @pl.when(k < SG // 2 - 1)
                def _():
                    gather(j0 + 2, buf_a)

                scat(j0 + 1, buf_b)
                return inner

            lax.fori_loop(0, SG // 2, body, 0)
            scat_wait(buf_b)
            return carry

        lax.fori_loop(0, CPS // SG, stage, 0)
        plsc.subcore_barrier()
        pltpu.sync_copy(acc.at[row], out_hbm.at[row])


def _prep_body(deg_ref, x_ref, dinv_ref, xs_ref):
    d = deg_ref[0] + deg_ref[1] + 1.0
    dinvb = lax.rsqrt(jnp.maximum(d, 1.0))
    dinv_ref[...] = dinvb
    xs_ref[...] = dinvb * x_ref[...]


def _prep_call(deg_parts, xpad):
    return pl.pallas_call(
        _prep_body,
        grid=(NP // RB,),
        in_specs=[
            pl.BlockSpec((NC, RB, D), lambda i: (0, i, 0)),
            pl.BlockSpec((RB, D), lambda i: (i, 0)),
        ],
        out_specs=[
            pl.BlockSpec((RB, D), lambda i: (i, 0)),
            pl.BlockSpec((RB, D), lambda i: (i, 0)),
        ],
        out_shape=[
            jax.ShapeDtypeStruct((NP, D), jnp.float32),
            jax.ShapeDtypeStruct((NP, D), jnp.float32),
        ],
    )(deg_parts, xpad)


def _lin1_body(p_ref, xs_ref, dinv_ref, w_ref, b_ref, h_ref, sh_ref):
    dinv = dinv_ref[...]
    pre = dinv * (p_ref[...] + xs_ref[...])
    h = jnp.dot(pre, w_ref[...], preferred_element_type=jnp.float32)
    h = jnp.maximum(h + b_ref[...], 0.0)
    h_ref[...] = h
    sh_ref[...] = dinv * h


def _lin1_call(p, xs, dinvb, W1, b1):
    return pl.pallas_call(
        _lin1_body,
        grid=(NP // RB,),
        in_specs=[
            pl.BlockSpec((RB, D), lambda i: (i, 0)),
            pl.BlockSpec((RB, D), lambda i: (i, 0)),
            pl.BlockSpec((RB, D), lambda i: (i, 0)),
            pl.BlockSpec((D, D), lambda i: (0, 0)),
            pl.BlockSpec((1, D), lambda i: (0, 0)),
        ],
        out_specs=[
            pl.BlockSpec((RB, D), lambda i: (i, 0)),
            pl.BlockSpec((RB, D), lambda i: (i, 0)),
        ],
        out_shape=[
            jax.ShapeDtypeStruct((NP, D), jnp.float32),
            jax.ShapeDtypeStruct((NP, D), jnp.float32),
        ],
    )(p, xs, dinvb, W1, b1)


def _lin2_body(p_ref, sh_ref, dinv_ref, w_ref, b_ref, out_ref):
    pre = dinv_ref[...] * (p_ref[...] + sh_ref[...])
    out = jnp.dot(pre, w_ref[...], preferred_element_type=jnp.float32)
    out_ref[...] = out + b_ref[...]


def _lin2_call(p, sh, dinvb, Wcat, bcat):
    return pl.pallas_call(
        _lin2_body,
        grid=(NP // RB,),
        in_specs=[
            pl.BlockSpec((RB, D), lambda i: (i, 0)),
            pl.BlockSpec((RB, D), lambda i: (i, 0)),
            pl.BlockSpec((RB, D), lambda i: (i, 0)),
            pl.BlockSpec((D, D), lambda i: (0, 0)),
            pl.BlockSpec((1, D), lambda i: (0, 0)),
        ],
        out_specs=pl.BlockSpec((RB, D), lambda i: (i, 0)),
        out_shape=jax.ShapeDtypeStruct((NP, D), jnp.float32),
    )(p, sh, dinvb, Wcat, bcat)


def kernel(x, edge_index, W1, b1, W_mu, b_mu, W_ls, b_ls):
    src = edge_index[0]
    dst = edge_index[1]
    pad = EP - E
    srcp = jnp.concatenate([src, jnp.zeros((pad,), jnp.int32)]).reshape(TOTCH, CB)
    dstp = jnp.concatenate([dst, jnp.full((pad,), TRASH, jnp.int32)]).reshape(TOTCH, CB)
    onesD = jnp.ones((CB, D), jnp.float32)
    zeroD = jnp.zeros((NP, D), jnp.float32)
    xpad = jnp.pad(x, ((0, NP - N), (0, 0)))

    deg_parts = _deg_kernel(dstp, onesD, zeroD)
    dinvb, xs = _prep_call(deg_parts, xpad)
    p1 = _agg_kernel(srcp, dstp, xs, zeroD)
    h, sh = _lin1_call(p1, xs, dinvb, W1, b1.reshape(1, D))
    p2 = _agg_kernel(srcp, dstp, sh, zeroD)
    Wcat = jnp.concatenate([W_mu, W_ls], axis=1)
    bcat = jnp.concatenate([b_mu, b_ls]).reshape(1, D)
    out = _lin2_call(p2, sh, dinvb, Wcat, bcat)
    return out[:N, :64], out[:N, 64:]


# speedup vs baseline: 1.3187x; 1.3187x over previous
import functools

import jax
import jax.numpy as jnp
from jax import lax
from jax.experimental import pallas as pl
from jax.experimental.pallas import tpu as pltpu
from jax.experimental.pallas import tpu_sc as plsc

N = 10000
E = 320000
D = 128

NC = 2
NS = 16
NW = NC * NS
CB = 128
CH = 80
EP = NW * CH * CB
TOTCH = EP // CB
FC = 1
CF = 120
CS = 40
SG = 40
NP = 10112
RPS = NP // NS
TRASH = N
RB = 2528

_mesh = plsc.VectorSubcoreMesh(core_axis_name="c", subcore_axis_name="s")


@functools.partial(
    pl.kernel,
    out_type=jax.ShapeDtypeStruct((NC, NP, D), jnp.float32),
    mesh=_mesh,
    scratch_types=[
        pltpu.VMEM((CH, CB), jnp.int32),
        pltpu.VMEM((CB, D), jnp.float32),
        pltpu.VMEM_SHARED((NP, D), jnp.float32),
        pltpu.SemaphoreType.DMA,
    ],
)
def _deg_kernel(dst_hbm, ones_hbm, zero_hbm, out_hbm, dst_v, ones_v, acc, sem):
    cid = lax.axis_index("c")
    sid = lax.axis_index("s")
    wid = sid * NC + cid
    row = pl.ds(sid * RPS, RPS)
    pltpu.sync_copy(zero_hbm.at[row], acc.at[row])
    pltpu.sync_copy(dst_hbm.at[pl.ds(wid * CH, CH)], dst_v)
    pltpu.sync_copy(ones_hbm, ones_v)
    plsc.subcore_barrier()

    GK = 8

    def body(g, carry):
        for i in range(GK):
            pltpu.async_copy(ones_v, acc.at[dst_v.at[g * GK + i]], sem, add=True)
        for i in range(GK):
            pltpu.make_async_copy(ones_v, acc.at[dst_v.at[g * GK + i]], sem).wait()
        return carry

    lax.fori_loop(0, CH // GK, body, 0)
    plsc.subcore_barrier()
    pltpu.sync_copy(acc.at[row], out_hbm.at[cid, row])


@functools.partial(
    pl.kernel,
    out_type=jax.ShapeDtypeStruct((NC, NP, D), jnp.float32),
    mesh=_mesh,
    scratch_types=[
        pltpu.VMEM((SG, CB), jnp.int32),
        pltpu.VMEM((SG, CB), jnp.int32),
        pltpu.VMEM((CB, D), jnp.float32),
        pltpu.VMEM((CB, D), jnp.float32),
        pltpu.VMEM_SHARED((NP, D), jnp.float32),
        pltpu.SemaphoreType.DMA,
        pltpu.SemaphoreType.DMA,
    ],
)
def _agg_kernel(src_hbm, dst_hbm, xs_hbm, zero_hbm, out_hbm,
                src_v, dst_v, buf_a, buf_b, acc, gsem, ssem):
    cid = lax.axis_index("c")
    sid = lax.axis_index("s")
    row = pl.ds(sid * RPS, RPS)
    base = jnp.where(cid == FC, sid * CF, 16 * CF + sid * CS)
    nstages = jnp.where(cid == FC, CF // SG, CS // SG)
    pltpu.sync_copy(zero_hbm.at[row], acc.at[row])
    plsc.subcore_barrier()

    def gather(j, buf):
        pltpu.async_copy(xs_hbm.at[src_v.at[j]], buf, gsem)

    def gather_wait(buf):
        pltpu.make_async_copy(xs_hbm.at[src_v.at[0]], buf, gsem).wait()

    def scat(j, buf):
        pltpu.async_copy(buf, acc.at[dst_v.at[j]], ssem, add=True)

    def scat_wait(buf):
        pltpu.make_async_copy(buf, acc.at[dst_v.at[0]], ssem).wait()

    def stage(h, carry):
        pltpu.sync_copy(src_hbm.at[pl.ds(base + h * SG, SG)], src_v)
        pltpu.sync_copy(dst_hbm.at[pl.ds(base + h * SG, SG)], dst_v)
        gather(0, buf_a)

        def body(k, inner):
            j0 = 2 * k
            gather_wait(buf_a)

            @pl.when(k > 0)
            def _():
                scat_wait(buf_b)

            gather(j0 + 1, buf_b)
            scat(j0, buf_a)
            gather_wait(buf_b)
            scat_wait(buf_a)

            @pl.when(k < SG // 2 - 1)
            def _():
                gather(j0 + 2, buf_a)

            scat(j0 + 1, buf_b)
            return inner

        lax.fori_loop(0, SG // 2, body, 0)
        scat_wait(buf_b)
        return carry

    lax.fori_loop(0, nstages, stage, 0)
    plsc.subcore_barrier()
    pltpu.sync_copy(acc.at[row], out_hbm.at[cid, row])


def _prep_body(deg_ref, x_ref, dinv_ref, xs_ref):
    d = deg_ref[0] + deg_ref[1] + 1.0
    dinvb = lax.rsqrt(jnp.maximum(d, 1.0))
    dinv_ref[...] = dinvb
    xs_ref[...] = dinvb * x_ref[...]


def _prep_call(deg_parts, xpad):
    return pl.pallas_call(
        _prep_body,
        grid=(NP // RB,),
        in_specs=[
            pl.BlockSpec((NC, RB, D), lambda i: (0, i, 0)),
            pl.BlockSpec((RB, D), lambda i: (i, 0)),
        ],
        out_specs=[
            pl.BlockSpec((RB, D), lambda i: (i, 0)),
            pl.BlockSpec((RB, D), lambda i: (i, 0)),
        ],
        out_shape=[
            jax.ShapeDtypeStruct((NP, D), jnp.float32),
            jax.ShapeDtypeStruct((NP, D), jnp.float32),
        ],
    )(deg_parts, xpad)


def _lin1_body(p_ref, xs_ref, dinv_ref, w_ref, b_ref, h_ref, sh_ref):
    dinv = dinv_ref[...]
    pre = dinv * (p_ref[0] + p_ref[1] + xs_ref[...])
    h = jnp.dot(pre, w_ref[...], preferred_element_type=jnp.float32)
    h = jnp.maximum(h + b_ref[...], 0.0)
    h_ref[...] = h
    sh_ref[...] = dinv * h


def _lin1_call(p, xs, dinvb, W1, b1):
    return pl.pallas_call(
        _lin1_body,
        grid=(NP // RB,),
        in_specs=[
            pl.BlockSpec((NC, RB, D), lambda i: (0, i, 0)),
            pl.BlockSpec((RB, D), lambda i: (i, 0)),
            pl.BlockSpec((RB, D), lambda i: (i, 0)),
            pl.BlockSpec((D, D), lambda i: (0, 0)),
            pl.BlockSpec((1, D), lambda i: (0, 0)),
        ],
        out_specs=[
            pl.BlockSpec((RB, D), lambda i: (i, 0)),
            pl.BlockSpec((RB, D), lambda i: (i, 0)),
        ],
        out_shape=[
            jax.ShapeDtypeStruct((NP, D), jnp.float32),
            jax.ShapeDtypeStruct((NP, D), jnp.float32),
        ],
    )(p, xs, dinvb, W1, b1)


def _lin2_body(p_ref, sh_ref, dinv_ref, w_ref, b_ref, out_ref):
    pre = dinv_ref[...] * (p_ref[0] + p_ref[1] + sh_ref[...])
    out = jnp.dot(pre, w_ref[...], preferred_element_type=jnp.float32)
    out_ref[...] = out + b_ref[...]


def _lin2_call(p, sh, dinvb, Wcat, bcat):
    return pl.pallas_call(
        _lin2_body,
        grid=(NP // RB,),
        in_specs=[
            pl.BlockSpec((NC, RB, D), lambda i: (0, i, 0)),
            pl.BlockSpec((RB, D), lambda i: (i, 0)),
            pl.BlockSpec((RB, D), lambda i: (i, 0)),
            pl.BlockSpec((D, D), lambda i: (0, 0)),
            pl.BlockSpec((1, D), lambda i: (0, 0)),
        ],
        out_specs=pl.BlockSpec((RB, D), lambda i: (i, 0)),
        out_shape=jax.ShapeDtypeStruct((NP, D), jnp.float32),
    )(p, sh, dinvb, Wcat, bcat)


def kernel(x, edge_index, W1, b1, W_mu, b_mu, W_ls, b_ls):
    src = edge_index[0]
    dst = edge_index[1]
    pad = EP - E
    srcp = jnp.concatenate([src, jnp.zeros((pad,), jnp.int32)]).reshape(TOTCH, CB)
    dstp = jnp.concatenate([dst, jnp.full((pad,), TRASH, jnp.int32)]).reshape(TOTCH, CB)
    onesD = jnp.ones((CB, D), jnp.float32)
    zeroD = jnp.zeros((NP, D), jnp.float32)
    xpad = jnp.pad(x, ((0, NP - N), (0, 0)))

    deg_parts = _deg_kernel(dstp, onesD, zeroD)
    dinvb, xs = _prep_call(deg_parts, xpad)
    p1 = _agg_kernel(srcp, dstp, xs, zeroD)
    h, sh = _lin1_call(p1, xs, dinvb, W1, b1.reshape(1, D))
    p2 = _agg_kernel(srcp, dstp, sh, zeroD)
    Wcat = jnp.concatenate([W_mu, W_ls], axis=1)
    bcat = jnp.concatenate([b_mu, b_ls]).reshape(1, D)
    out = _lin2_call(p2, sh, dinvb, Wcat, bcat)
    return out[:N, :64], out[:N, 64:]
